# baseline (speedup 1.0000x reference)
import functools

import jax
import jax.numpy as jnp
from jax import lax
from jax.experimental import pallas as pl
from jax.experimental.pallas import tpu as pltpu
from jax.experimental.pallas import tpu_sc as plsc

EPS = 1e-5
F32 = jnp.float32


def _dotT(a, b):
    return lax.dot_general(a, b, (((1,), (1,)), ((), ())),
                           preferred_element_type=F32)


def _k1a_body(N, RPB, x_ref, wp_ref, knn_ref, p_ref, adj_ref):
    p_ref[...] = _dotT(x_ref[...], wp_ref[...])

    @pl.when(pl.program_id(0) == 0)
    def _():
        kr = adj_ref.shape[0]
        b = lax.broadcasted_iota(jnp.int32, (kr, 128), 0) // RPB
        adj_ref[...] = knn_ref[...] + b * N


def _run_k1a(xr, w_proj, knn2, N, rb):
    r, c = xr.shape
    grid = (r // rb,)
    kr = knn2.shape[0]
    rpb = kr * N // r
    row_spec = pl.BlockSpec((rb, c), lambda i: (i, 0))
    w_spec = pl.BlockSpec((c, c), lambda i: (0, 0))
    k_spec = pl.BlockSpec((kr, 128), lambda i: (0, 0))
    return pl.pallas_call(
        functools.partial(_k1a_body, N, rpb),
        grid=grid,
        in_specs=[row_spec, w_spec, k_spec],
        out_specs=[row_spec, k_spec],
        out_shape=[jax.ShapeDtypeStruct((r, c), F32),
                   jax.ShapeDtypeStruct((kr, 128), jnp.int32)],
    )(xr, w_proj, knn2)


def _k1b_body(x_ref, wg_ref, y2_ref):
    y2_ref[...] = _dotT(x_ref[...], wg_ref[...])


def _run_k1b(xr, w_global, rb):
    r, c = xr.shape
    grid = (r // rb,)
    row_spec = pl.BlockSpec((rb, c), lambda i: (i, 0))
    w_spec = pl.BlockSpec((c, c), lambda i: (0, 0))
    return pl.pallas_call(
        _k1b_body,
        grid=grid,
        in_specs=[row_spec, w_spec],
        out_specs=row_spec,
        out_shape=jax.ShapeDtypeStruct((r, c), F32),
    )(xr, w_global)


def _make_sc_gather_max(ROWS, ROW_LO, C, K):
    NC, L = 2, 16
    NW = NC * 16
    SUB = 8
    IDXB = SUB * K
    NPR = ROWS // 16
    PBASE = NPR // NW
    PXTRA = NPR - PBASE * NW
    assert ROWS % 16 == 0 and PXTRA < NW and PBASE >= 2
    CB = C // L
    assert K == L

    mesh = plsc.VectorSubcoreMesh(core_axis_name="c", subcore_axis_name="s")

    @functools.partial(
        pl.kernel, mesh=mesh,
        out_type=jax.ShapeDtypeStruct((ROWS, C), F32),
        scratch_types=[
            pltpu.VMEM((2 * (PBASE + 1) * IDXB,), jnp.int32),
            pltpu.VMEM((IDXB, C), F32),
            pltpu.VMEM((IDXB, C), F32),
            pltpu.VMEM((SUB, C), F32),
            pltpu.VMEM((SUB, C), F32),
            pltpu.SemaphoreType.DMA,
            pltpu.SemaphoreType.DMA,
            pltpu.SemaphoreType.DMA,
            pltpu.SemaphoreType.DMA,
        ],
    )
    def sc_gather_max(p_hbm, adj_hbm, out_hbm, idx_v, g0, g1, o0, o1,
                      sg0, sg1, so0, so1):
        wid = lax.axis_index("s") * NC + lax.axis_index("c")
        npair = PBASE + jnp.where(wid < PXTRA, 1, 0)
        row0 = 16 * (PBASE * wid + jnp.minimum(wid, PXTRA))

        def gather(u, gbuf, sem):
            idx_sl = idx_v.at[pl.ds(u * IDXB, IDXB)]
            return pltpu.make_async_copy(p_hbm.at[idx_sl], gbuf, sem)

        def out_copy(u, obuf, sem):
            dst = out_hbm.at[pl.ds(row0 + u * SUB, SUB)]
            return pltpu.make_async_copy(obuf, dst, sem)

        def compute(u, gbuf, obuf):
            def row_body(rr, _):
                for cb in range(CB):
                    sl = pl.ds(cb * L, L)
                    m = gbuf[rr * K, sl]
                    for kk in range(1, K):
                        m = jnp.maximum(m, gbuf[rr * K + kk, sl])
                    obuf[rr, sl] = m
                return 0

            lax.fori_loop(0, SUB, row_body, 0)

        src0 = (ROW_LO + row0) * K
        pltpu.sync_copy(adj_hbm.at[pl.ds(src0, 2 * PBASE * IDXB)],
                        idx_v.at[pl.ds(0, 2 * PBASE * IDXB)])

        @pl.when(wid < PXTRA)
        def _():
            pltpu.sync_copy(
                adj_hbm.at[pl.ds(src0 + 2 * PBASE * IDXB, 2 * IDXB)],
                idx_v.at[pl.ds(2 * PBASE * IDXB, 2 * IDXB)])

        gather(0, g0, sg0).start()

        def pair_body(t, _):
            q0 = 2 * t
            gather(q0 + 1, g1, sg1).start()
            gather(q0, g0, sg0).wait()

            @pl.when(t > 0)
            def _():
                out_copy(q0 - 2, o0, so0).wait()

            compute(q0, g0, o0)
            out_copy(q0, o0, so0).start()

            @pl.when(t < npair - 1)
            def _():
                gather(q0 + 2, g0, sg0).start()

            gather(q0 + 1, g1, sg1).wait()

            @pl.when(t > 0)
            def _():
                out_copy(q0 - 1, o1, so1).wait()

            compute(q0 + 1, g1, o1)
            out_copy(q0 + 1, o1, so1).start()
            return 0

        lax.fori_loop(0, npair, pair_body, 0)

        out_copy(2 * npair - 2, o0, so0).wait()
        out_copy(2 * npair - 1, o1, so1).wait()

    return sc_gather_max


def _derive_coef(R, C, s, G, A1, A2, v_ref, coef_ref):
    gl, bl = v_ref[0:1, :], v_ref[1:2, :]
    gg, bg = v_ref[2:3, :], v_ref[3:4, :]
    ab1, ag1, abt1 = v_ref[4:5, :], v_ref[5:6, :], v_ref[6:7, :]
    ab2, ag2, abt2 = v_ref[7:8, :], v_ref[8:9, :], v_ref[9:10, :]

    mu = s / R
    outer = lax.dot_general(mu, mu, (((0,), (0,)), ((), ())),
                            preferred_element_type=F32)
    cov = G / R - outer
    i0 = lax.broadcasted_iota(jnp.int32, (2 * C, 2 * C), 0)
    i1 = lax.broadcasted_iota(jnp.int32, (2 * C, 2 * C), 1)
    diag = jnp.sum(jnp.where(i0 == i1, G, 0.0), axis=0,
                   keepdims=True) / R
    var_z = diag - mu * mu
    a1 = gl * lax.rsqrt(var_z[:, :C] + EPS)
    c1 = bl - a1 * mu[:, :C]
    a2 = gg * lax.rsqrt(var_z[:, C:] + EPS)
    c2 = bg - a2 * mu[:, C:]

    mut1 = _dotT(bl + bg, A1) + ab1
    M1 = jnp.concatenate([A1 * a1, A1 * a2], axis=1)
    M1cov = lax.dot_general(M1, cov, (((1,), (0,)), ((), ())),
                            preferred_element_type=F32)
    vart1 = jnp.sum(M1cov * M1, axis=1).reshape(1, C)
    al1 = ag1 * lax.rsqrt(vart1 + EPS)
    g1c = abt1 - al1 * mut1

    mut2 = _dotT(abt1, A2) + ab2
    M2 = lax.dot_general(A2 * al1, M1, (((1,), (0,)), ((), ())),
                         preferred_element_type=F32)
    M2cov = lax.dot_general(M2, cov, (((1,), (0,)), ((), ())),
                            preferred_element_type=F32)
    vart2 = jnp.sum(M2cov * M2, axis=1).reshape(1, C)
    al2 = ag2 * lax.rsqrt(vart2 + EPS)
    g2c = abt2 - al2 * mut2

    coef_ref[0:1, :] = a1
    coef_ref[1:2, :] = c1
    coef_ref[2:3, :] = a2
    coef_ref[3:4, :] = c2
    coef_ref[4:5, :] = ab1
    coef_ref[5:6, :] = al1
    coef_ref[6:7, :] = g1c
    coef_ref[7:8, :] = ab2
    coef_ref[8:9, :] = al2
    coef_ref[9:10, :] = g2c


def _ks_body(xk_ref, p_ref, y2_ref, wl_ref, y1_ref, s_ref, g_ref):
    y1 = _dotT(xk_ref[...] - p_ref[...], wl_ref[...])
    y1_ref[...] = y1
    z = jnp.concatenate([y1, y2_ref[...]], axis=1)

    @pl.when(pl.program_id(0) == 0)
    def _():
        s_ref[...] = jnp.zeros_like(s_ref)
        g_ref[...] = jnp.zeros_like(g_ref)

    s_ref[...] += jnp.sum(z, axis=0, keepdims=True)
    g_ref[...] += lax.dot_general(z, z, (((0,), (0,)), ((), ())),
                                  preferred_element_type=F32)


def _run_ks(xk_h, p, y2, w_local, off, rb):
    rows, c = xk_h.shape
    ob = off // rb
    grid = (rows // rb,)
    h_spec = pl.BlockSpec((rb, c), lambda i: (i, 0))
    f_spec = pl.BlockSpec((rb, c), lambda i: (i + ob, 0))
    w_spec = pl.BlockSpec((c, c), lambda i: (0, 0))
    return pl.pallas_call(
        _ks_body,
        grid=grid,
        in_specs=[h_spec, f_spec, f_spec, w_spec],
        out_specs=[h_spec,
                   pl.BlockSpec((1, 2 * c), lambda i: (0, 0)),
                   pl.BlockSpec((2 * c, 2 * c), lambda i: (0, 0))],
        out_shape=[jax.ShapeDtypeStruct((rows, c), F32),
                   jax.ShapeDtypeStruct((1, 2 * c), F32),
                   jax.ShapeDtypeStruct((2 * c, 2 * c), F32)],
    )(xk_h, p, y2, w_local)


def _k5_body(R, C, NB, NA, RB, y1a_ref, y1b_ref, y2_ref, a1w_ref, a2w_ref,
             v_ref, gb_ref, s1_ref, s2_ref, g1_ref, g2_ref, res_ref,
             o_s, coef_s, so_s, soq_s):
    i = pl.program_id(0)

    @pl.when(i == 0)
    def _():
        _derive_coef(float(R), C, s1_ref[...] + s2_ref[...],
                     g1_ref[...] + g2_ref[...], a1w_ref[...],
                     a2w_ref[...], v_ref, coef_s)
        so_s[...] = jnp.zeros_like(so_s)
        soq_s[...] = jnp.zeros_like(soq_s)

    def attn(y1):
        y2 = y2_ref[...]
        cf = coef_s[...]
        a1, c1 = cf[0:1, :], cf[1:2, :]
        a2, c2 = cf[2:3, :], cf[3:4, :]
        b1, al1, g1c = cf[4:5, :], cf[5:6, :], cf[6:7, :]
        b2, al2, g2c = cf[7:8, :], cf[8:9, :], cf[9:10, :]
        x1 = y1 * a1 + c1
        x2 = y2 * a2 + c2
        t1 = _dotT(x1 + x2, a1w_ref[...]) + b1
        u1 = t1 * al1 + g1c
        t2 = _dotT(u1, a2w_ref[...]) + b2
        att = jax.nn.sigmoid(t2 * al2 + g2c)
        o = x2 + att * (x1 - x2)
        o_s[pl.ds(i * RB, RB), :] = o
        so_s[...] += jnp.sum(o, axis=0, keepdims=True)
        soq_s[...] += jnp.sum(o * o, axis=0, keepdims=True)

    @pl.when(i < NA)
    def _():
        attn(y1a_ref[...])

    @pl.when(jnp.logical_and(i >= NA, i < NB))
    def _():
        attn(y1b_ref[...])

    @pl.when(i >= NB)
    def _():
        j = i - NB
        mu = so_s[...] / R
        var = soq_s[...] / R - mu * mu
        scale = gb_ref[0:1, :] * lax.rsqrt(var + EPS)
        res_ref[...] = (o_s[pl.ds(j * RB, RB), :] - mu) * scale \
            + gb_ref[1:2, :]


def _run_k5(y1a, y1b, y2, aff_w1, aff_w2, vecs, gb, s1, s2, G1, G2, rb):
    r, c = y2.shape
    nb = r // rb
    na = y1a.shape[0] // rb
    nbb = y1b.shape[0] // rb
    a_spec = pl.BlockSpec((rb, c), lambda i: (jnp.minimum(i, na - 1), 0))
    b_spec = pl.BlockSpec(
        (rb, c),
        lambda i: (jnp.clip(i - na, 0, nbb - 1), 0))
    y2_spec = pl.BlockSpec((rb, c), lambda i: (jnp.minimum(i, nb - 1), 0))
    row3 = pl.BlockSpec((rb, c), lambda i: (jnp.maximum(i - nb, 0), 0))
    w_spec = pl.BlockSpec((c, c), lambda i: (0, 0))
    s_spec = pl.BlockSpec((1, 2 * c), lambda i: (0, 0))
    g_spec = pl.BlockSpec((2 * c, 2 * c), lambda i: (0, 0))
    return pl.pallas_call(
        functools.partial(_k5_body, r, c, nb, na, rb),
        grid=(2 * nb,),
        in_specs=[a_spec, b_spec, y2_spec, w_spec, w_spec,
                  pl.BlockSpec((10, c), lambda i: (0, 0)),
                  pl.BlockSpec((2, c), lambda i: (0, 0)),
                  s_spec, s_spec, g_spec, g_spec],
        out_specs=row3,
        out_shape=jax.ShapeDtypeStruct((r, c), F32),
        scratch_shapes=[
            pltpu.VMEM((r, c), F32),
            pltpu.VMEM((16, c), F32),
            pltpu.VMEM((1, c), F32),
            pltpu.VMEM((1, c), F32),
        ],
    )(y1a, y1b, y2, aff_w1, aff_w2, vecs, gb, s1, s2, G1, G2)


def kernel(x, knn, w_proj, w_local, g_local, b_local, w_global, g_global,
           b_global, aff_w1, aff_b1, aff_g1, aff_bt1, aff_w2, aff_b2,
           aff_g2, aff_bt2, bn_g, bn_b):
    B, N, C = x.shape
    K = knn.shape[-1]
    R = B * N
    RB = 2000
    H = R // 2

    xr = x.reshape(R, C)
    knn2 = knn.reshape(R * K // 128, 128)
    p, adj = _run_k1a(xr, w_proj, knn2, N, RB)
    adjf = adj.reshape(R * K)

    xk_a = _make_sc_gather_max(H, 0, C, K)(p, adjf)
    y2 = _run_k1b(xr, w_global, RB)
    xk_b = _make_sc_gather_max(H, H, C, K)(p, adjf)
    y1a, s1, G1 = _run_ks(xk_a, p, y2, w_local, 0, RB)
    y1b, s2, G2 = _run_ks(xk_b, p, y2, w_local, H, RB)

    vecs = jnp.stack([g_local, b_local, g_global, b_global,
                      aff_b1, aff_g1, aff_bt1,
                      aff_b2, aff_g2, aff_bt2], axis=0)
    gb = jnp.stack([bn_g, bn_b], axis=0)
    res = _run_k5(y1a, y1b, y2, aff_w1, aff_w2, vecs, gb,
                  s1, s2, G1, G2, RB)
    return res.reshape(B, N, C)


# speedup vs baseline: 1.0248x; 1.0248x over previous
import functools

import jax
import jax.numpy as jnp
from jax import lax
from jax.experimental import pallas as pl
from jax.experimental.pallas import tpu as pltpu
from jax.experimental.pallas import tpu_sc as plsc

EPS = 1e-5
F32 = jnp.float32


def _dotT(a, b):
    return lax.dot_general(a, b, (((1,), (1,)), ((), ())),
                           preferred_element_type=F32)


def _k1_body(N, RPB, x_ref, wp_ref, wg_ref, knn_ref, p_ref, y2_ref, adj_ref):
    xb = x_ref[...]
    p_ref[...] = _dotT(xb, wp_ref[...])
    y2_ref[...] = _dotT(xb, wg_ref[...])

    @pl.when(pl.program_id(0) == 0)
    def _():
        kr = adj_ref.shape[0]
        b = lax.broadcasted_iota(jnp.int32, (kr, 128), 0) // RPB
        adj_ref[...] = knn_ref[...] + b * N


def _run_k1(xr, w_proj, w_global, knn2, N, rb):
    r, c = xr.shape
    grid = (r // rb,)
    kr = knn2.shape[0]
    rpb = kr * N * c // (r * c)
    row_spec = pl.BlockSpec((rb, c), lambda i: (i, 0))
    w_spec = pl.BlockSpec((c, c), lambda i: (0, 0))
    k_spec = pl.BlockSpec((kr, 128), lambda i: (0, 0))
    return pl.pallas_call(
        functools.partial(_k1_body, N, rpb),
        grid=grid,
        in_specs=[row_spec, w_spec, w_spec, k_spec],
        out_specs=[row_spec, row_spec, k_spec],
        out_shape=[jax.ShapeDtypeStruct((r, c), F32),
                   jax.ShapeDtypeStruct((r, c), F32),
                   jax.ShapeDtypeStruct((kr, 128), jnp.int32)],
    )(xr, w_proj, w_global, knn2)


def _make_sc_gather_max(R, C, K):
    NC, L = 2, 16
    NW = NC * 16
    NBLK = R // 8
    BASE = NBLK // NW
    XTRA = NBLK - BASE * NW
    SUB = 8
    IDXB = SUB * K
    NPAIR = BASE // 2
    assert BASE % 2 == 0 and XTRA < NW
    CB = C // L
    assert K == L

    mesh = plsc.VectorSubcoreMesh(core_axis_name="c", subcore_axis_name="s")

    @functools.partial(
        pl.kernel, mesh=mesh,
        out_type=jax.ShapeDtypeStruct((R, C), F32),
        scratch_types=[
            pltpu.VMEM(((BASE + 1) * IDXB,), jnp.int32),
            pltpu.VMEM((IDXB, C), F32),
            pltpu.VMEM((IDXB, C), F32),
            pltpu.VMEM((IDXB, C), F32),
            pltpu.VMEM((SUB, C), F32),
            pltpu.VMEM((SUB, C), F32),
            pltpu.SemaphoreType.DMA,
            pltpu.SemaphoreType.DMA,
            pltpu.SemaphoreType.DMA,
            pltpu.SemaphoreType.DMA,
            pltpu.SemaphoreType.DMA,
        ],
    )
    def sc_gather_max(p_hbm, adj_hbm, out_hbm, idx_v, g0, g1, g2, o0, o1,
                      sg0, sg1, sg2, so0, so1):
        wid = lax.axis_index("s") * NC + lax.axis_index("c")
        row0 = 8 * (BASE * wid + jnp.minimum(wid, XTRA))
        has_tail = wid < XTRA

        def gather(u, gbuf, sem):
            idx_sl = idx_v.at[pl.ds(u * IDXB, IDXB)]
            return pltpu.make_async_copy(p_hbm.at[idx_sl], gbuf, sem)

        def out_copy(u, obuf, sem):
            dst = out_hbm.at[pl.ds(row0 + u * SUB, SUB)]
            return pltpu.make_async_copy(obuf, dst, sem)

        def compute(u, gbuf, obuf):
            def row_body(rr, _):
                for cb in range(CB):
                    sl = pl.ds(cb * L, L)
                    vals = [gbuf[rr * K + kk, sl] for kk in range(K)]
                    while len(vals) > 1:
                        vals = [jnp.maximum(vals[j], vals[j + 1])
                                for j in range(0, len(vals), 2)]
                    obuf[rr, sl] = vals[0]
                return 0

            lax.fori_loop(0, SUB, row_body, 0)

        pltpu.sync_copy(adj_hbm.at[pl.ds(row0 * K, BASE * IDXB)],
                        idx_v.at[pl.ds(0, BASE * IDXB)])

        @pl.when(has_tail)
        def _():
            pltpu.sync_copy(adj_hbm.at[pl.ds(row0 * K + BASE * IDXB, IDXB)],
                            idx_v.at[pl.ds(BASE * IDXB, IDXB)])
            gather(BASE, g2, sg2).start()

        gather(0, g0, sg0).start()

        def pair_body(t, _):
            q0 = 2 * t
            gather(q0 + 1, g1, sg1).start()
            gather(q0, g0, sg0).wait()

            @pl.when(t > 0)
            def _():
                out_copy(q0 - 2, o0, so0).wait()

            compute(q0, g0, o0)
            out_copy(q0, o0, so0).start()

            @pl.when(t < NPAIR - 1)
            def _():
                gather(q0 + 2, g0, sg0).start()

            gather(q0 + 1, g1, sg1).wait()

            @pl.when(t > 0)
            def _():
                out_copy(q0 - 1, o1, so1).wait()

            compute(q0 + 1, g1, o1)
            out_copy(q0 + 1, o1, so1).start()
            return 0

        lax.fori_loop(0, NPAIR, pair_body, 0)

        @pl.when(has_tail)
        def _():
            gather(BASE, g2, sg2).wait()
            out_copy(BASE - 2, o0, so0).wait()
            compute(BASE, g2, o0)
            out_copy(BASE, o0, so0).start()
            out_copy(BASE, o0, so0).wait()

        @pl.when(wid >= XTRA)
        def _():
            out_copy(BASE - 2, o0, so0).wait()

        out_copy(BASE - 1, o1, so1).wait()

    return sc_gather_max


def _derive_coef(R, C, s, G, A1, A2, v_ref, coef_ref):
    gl, bl = v_ref[0:1, :], v_ref[1:2, :]
    gg, bg = v_ref[2:3, :], v_ref[3:4, :]
    ab1, ag1, abt1 = v_ref[4:5, :], v_ref[5:6, :], v_ref[6:7, :]
    ab2, ag2, abt2 = v_ref[7:8, :], v_ref[8:9, :], v_ref[9:10, :]

    mu = s / R
    outer = lax.dot_general(mu, mu, (((0,), (0,)), ((), ())),
                            preferred_element_type=F32)
    cov = G / R - outer
    i0 = lax.broadcasted_iota(jnp.int32, (2 * C, 2 * C), 0)
    i1 = lax.broadcasted_iota(jnp.int32, (2 * C, 2 * C), 1)
    diag = jnp.sum(jnp.where(i0 == i1, G, 0.0), axis=0,
                   keepdims=True) / R
    var_z = diag - mu * mu
    a1 = gl * lax.rsqrt(var_z[:, :C] + EPS)
    c1 = bl - a1 * mu[:, :C]
    a2 = gg * lax.rsqrt(var_z[:, C:] + EPS)
    c2 = bg - a2 * mu[:, C:]

    mut1 = _dotT(bl + bg, A1) + ab1
    M1 = jnp.concatenate([A1 * a1, A1 * a2], axis=1)
    M1cov = lax.dot_general(M1, cov, (((1,), (0,)), ((), ())),
                            preferred_element_type=F32)
    vart1 = jnp.sum(M1cov * M1, axis=1).reshape(1, C)
    al1 = ag1 * lax.rsqrt(vart1 + EPS)
    g1c = abt1 - al1 * mut1

    mut2 = _dotT(abt1, A2) + ab2
    M2 = lax.dot_general(A2 * al1, M1, (((1,), (0,)), ((), ())),
                         preferred_element_type=F32)
    M2cov = lax.dot_general(M2, cov, (((1,), (0,)), ((), ())),
                            preferred_element_type=F32)
    vart2 = jnp.sum(M2cov * M2, axis=1).reshape(1, C)
    al2 = ag2 * lax.rsqrt(vart2 + EPS)
    g2c = abt2 - al2 * mut2

    coef_ref[0:1, :] = a1
    coef_ref[1:2, :] = c1
    coef_ref[2:3, :] = a2
    coef_ref[3:4, :] = c2
    coef_ref[4:5, :] = ab1
    coef_ref[5:6, :] = al1
    coef_ref[6:7, :] = g1c
    coef_ref[7:8, :] = ab2
    coef_ref[8:9, :] = al2
    coef_ref[9:10, :] = g2c


def _k5_body(R, C, NB, RB, xk_ref, p_ref, y2_ref, wl_ref, a1w_ref, a2w_ref,
             v_ref, gb_ref, res_ref, y1_s, o_s, s_s, g_s, coef_s,
             so_s, soq_s):
    i = pl.program_id(0)

    @pl.when(i < NB)
    def _():
        y1 = _dotT(xk_ref[...] - p_ref[...], wl_ref[...])
        y1_s[pl.ds(i * RB, RB), :] = y1
        z = jnp.concatenate([y1, y2_ref[...]], axis=1)

        @pl.when(i == 0)
        def _():
            s_s[...] = jnp.zeros_like(s_s)
            g_s[...] = jnp.zeros_like(g_s)

        s_s[...] += jnp.sum(z, axis=0, keepdims=True)
        g_s[...] += lax.dot_general(z, z, (((0,), (0,)), ((), ())),
                                    preferred_element_type=F32)

    @pl.when(i == NB)
    def _():
        _derive_coef(float(R), C, s_s[...], g_s[...], a1w_ref[...],
                     a2w_ref[...], v_ref, coef_s)

    @pl.when(jnp.logical_and(i >= NB, i < 2 * NB))
    def _():
        j = i - NB
        y1 = y1_s[pl.ds(j * RB, RB), :]
        y2 = y2_ref[...]
        cf = coef_s[...]
        a1, c1 = cf[0:1, :], cf[1:2, :]
        a2, c2 = cf[2:3, :], cf[3:4, :]
        b1, al1, g1c = cf[4:5, :], cf[5:6, :], cf[6:7, :]
        b2, al2, g2c = cf[7:8, :], cf[8:9, :], cf[9:10, :]

        x1 = y1 * a1 + c1
        x2 = y2 * a2 + c2
        t1 = _dotT(x1 + x2, a1w_ref[...]) + b1
        u1 = t1 * al1 + g1c
        t2 = _dotT(u1, a2w_ref[...]) + b2
        att = jax.nn.sigmoid(t2 * al2 + g2c)
        o = x2 + att * (x1 - x2)
        o_s[pl.ds(j * RB, RB), :] = o

        @pl.when(i == NB)
        def _():
            so_s[...] = jnp.zeros_like(so_s)
            soq_s[...] = jnp.zeros_like(soq_s)

        so_s[...] += jnp.sum(o, axis=0, keepdims=True)
        soq_s[...] += jnp.sum(o * o, axis=0, keepdims=True)

    @pl.when(i >= 2 * NB)
    def _():
        j = i - 2 * NB
        mu = so_s[...] / R
        var = soq_s[...] / R - mu * mu
        scale = gb_ref[0:1, :] * lax.rsqrt(var + EPS)
        res_ref[...] = (o_s[pl.ds(j * RB, RB), :] - mu) * scale \
            + gb_ref[1:2, :]


def _run_k5(xk, p, y2, w_local, aff_w1, aff_w2, vecs, gb, rb):
    r, c = xk.shape
    nb = r // rb
    row1 = pl.BlockSpec((rb, c), lambda i: (jnp.minimum(i, nb - 1), 0))
    row12 = pl.BlockSpec(
        (rb, c),
        lambda i: (jnp.where(i < nb, i, jnp.minimum(i - nb, nb - 1)), 0))
    row3 = pl.BlockSpec((rb, c), lambda i: (jnp.maximum(i - 2 * nb, 0), 0))
    w_spec = pl.BlockSpec((c, c), lambda i: (0, 0))
    return pl.pallas_call(
        functools.partial(_k5_body, r, c, nb, rb),
        grid=(3 * nb,),
        in_specs=[row1, row1, row12, w_spec, w_spec, w_spec,
                  pl.BlockSpec((10, c), lambda i: (0, 0)),
                  pl.BlockSpec((2, c), lambda i: (0, 0))],
        out_specs=row3,
        out_shape=jax.ShapeDtypeStruct((r, c), F32),
        scratch_shapes=[
            pltpu.VMEM((r, c), F32),
            pltpu.VMEM((r, c), F32),
            pltpu.VMEM((1, 2 * c), F32),
            pltpu.VMEM((2 * c, 2 * c), F32),
            pltpu.VMEM((16, c), F32),
            pltpu.VMEM((1, c), F32),
            pltpu.VMEM((1, c), F32),
        ],
    )(xk, p, y2, w_local, aff_w1, aff_w2, vecs, gb)


def kernel(x, knn, w_proj, w_local, g_local, b_local, w_global, g_global,
           b_global, aff_w1, aff_b1, aff_g1, aff_bt1, aff_w2, aff_b2,
           aff_g2, aff_bt2, bn_g, bn_b):
    B, N, C = x.shape
    K = knn.shape[-1]
    R = B * N
    RB = 2000

    xr = x.reshape(R, C)
    knn2 = knn.reshape(R * K // 128, 128)
    p, y2, adj = _run_k1(xr, w_proj, w_global, knn2, N, RB)

    sc = _make_sc_gather_max(R, C, K)
    xkmax = sc(p, adj.reshape(R * K))

    vecs = jnp.stack([g_local, b_local, g_global, b_global,
                      aff_b1, aff_g1, aff_bt1,
                      aff_b2, aff_g2, aff_bt2], axis=0)
    gb = jnp.stack([bn_g, bn_b], axis=0)
    res = _run_k5(xkmax, p, y2, w_local, aff_w1, aff_w2, vecs, gb, RB)
    return res.reshape(B, N, C)


# speedup vs baseline: 1.0450x; 1.0196x over previous
import functools

import jax
import jax.numpy as jnp
from jax import lax
from jax.experimental import pallas as pl
from jax.experimental.pallas import tpu as pltpu
from jax.experimental.pallas import tpu_sc as plsc

EPS = 1e-5
F32 = jnp.float32


def _dotT(a, b):
    return lax.dot_general(a, b, (((1,), (1,)), ((), ())),
                           preferred_element_type=F32)


def _k1_body(N, RPB, x_ref, wp_ref, wg_ref, knn_ref, p_ref, y2_ref, adj_ref):
    xb = x_ref[...]
    p_ref[...] = _dotT(xb, wp_ref[...])
    y2_ref[...] = _dotT(xb, wg_ref[...])

    @pl.when(pl.program_id(0) == 0)
    def _():
        kr = adj_ref.shape[0]
        b = lax.broadcasted_iota(jnp.int32, (kr, 128), 0) // RPB
        adj_ref[...] = knn_ref[...] + b * N


def _run_k1(xr, w_proj, w_global, knn2, N, rb):
    r, c = xr.shape
    grid = (r // rb,)
    kr = knn2.shape[0]
    rpb = kr * N * c // (r * c)
    row_spec = pl.BlockSpec((rb, c), lambda i: (i, 0))
    w_spec = pl.BlockSpec((c, c), lambda i: (0, 0))
    k_spec = pl.BlockSpec((kr, 128), lambda i: (0, 0))
    return pl.pallas_call(
        functools.partial(_k1_body, N, rpb),
        grid=grid,
        in_specs=[row_spec, w_spec, w_spec, k_spec],
        out_specs=[row_spec, row_spec, k_spec],
        out_shape=[jax.ShapeDtypeStruct((r, c), F32),
                   jax.ShapeDtypeStruct((r, c), F32),
                   jax.ShapeDtypeStruct((kr, 128), jnp.int32)],
    )(xr, w_proj, w_global, knn2)


def _make_sc_gather_max(R, C, K):
    NC, L = 2, 16
    NW = NC * 16
    NBLK = R // 8
    BASE = NBLK // NW
    XTRA = NBLK - BASE * NW
    SUB = 8
    IDXB = SUB * K
    NPAIR = BASE // 2
    assert BASE % 2 == 0 and XTRA < NW
    CB = C // L
    assert K == L

    mesh = plsc.VectorSubcoreMesh(core_axis_name="c", subcore_axis_name="s")

    @functools.partial(
        pl.kernel, mesh=mesh,
        out_type=jax.ShapeDtypeStruct((R, C), F32),
        scratch_types=[
            pltpu.VMEM(((BASE + 1) * IDXB,), jnp.int32),
            pltpu.VMEM((IDXB, C), F32),
            pltpu.VMEM((IDXB, C), F32),
            pltpu.VMEM((IDXB, C), F32),
            pltpu.VMEM((SUB, C), F32),
            pltpu.VMEM((SUB, C), F32),
            pltpu.SemaphoreType.DMA,
            pltpu.SemaphoreType.DMA,
            pltpu.SemaphoreType.DMA,
            pltpu.SemaphoreType.DMA,
            pltpu.SemaphoreType.DMA,
        ],
    )
    def sc_gather_max(p_hbm, adj_hbm, out_hbm, idx_v, g0, g1, g2, o0, o1,
                      sg0, sg1, sg2, so0, so1):
        wid = lax.axis_index("s") * NC + lax.axis_index("c")
        row0 = 8 * (BASE * wid + jnp.minimum(wid, XTRA))
        has_tail = wid < XTRA

        def gather(u, gbuf, sem):
            idx_sl = idx_v.at[pl.ds(u * IDXB, IDXB)]
            return pltpu.make_async_copy(p_hbm.at[idx_sl], gbuf, sem)

        def out_copy(u, obuf, sem):
            dst = out_hbm.at[pl.ds(row0 + u * SUB, SUB)]
            return pltpu.make_async_copy(obuf, dst, sem)

        def compute(u, gbuf, obuf):
            def row_body(rr, _):
                for cb in range(CB):
                    sl = pl.ds(cb * L, L)
                    m = gbuf[rr * K, sl]
                    for kk in range(1, K):
                        m = jnp.maximum(m, gbuf[rr * K + kk, sl])
                    obuf[rr, sl] = m
                return 0

            lax.fori_loop(0, SUB, row_body, 0)

        pltpu.sync_copy(adj_hbm.at[pl.ds(row0 * K, BASE * IDXB)],
                        idx_v.at[pl.ds(0, BASE * IDXB)])

        @pl.when(has_tail)
        def _():
            pltpu.sync_copy(adj_hbm.at[pl.ds(row0 * K + BASE * IDXB, IDXB)],
                            idx_v.at[pl.ds(BASE * IDXB, IDXB)])
            gather(BASE, g2, sg2).start()

        gather(0, g0, sg0).start()

        def pair_body(t, _):
            q0 = 2 * t
            gather(q0 + 1, g1, sg1).start()
            gather(q0, g0, sg0).wait()

            @pl.when(t > 0)
            def _():
                out_copy(q0 - 2, o0, so0).wait()

            compute(q0, g0, o0)
            out_copy(q0, o0, so0).start()

            @pl.when(t < NPAIR - 1)
            def _():
                gather(q0 + 2, g0, sg0).start()

            gather(q0 + 1, g1, sg1).wait()

            @pl.when(t > 0)
            def _():
                out_copy(q0 - 1, o1, so1).wait()

            compute(q0 + 1, g1, o1)
            out_copy(q0 + 1, o1, so1).start()
            return 0

        lax.fori_loop(0, NPAIR, pair_body, 0)

        @pl.when(has_tail)
        def _():
            gather(BASE, g2, sg2).wait()
            out_copy(BASE - 2, o0, so0).wait()
            compute(BASE, g2, o0)
            out_copy(BASE, o0, so0).start()
            out_copy(BASE, o0, so0).wait()

        @pl.when(wid >= XTRA)
        def _():
            out_copy(BASE - 2, o0, so0).wait()

        out_copy(BASE - 1, o1, so1).wait()

    return sc_gather_max


def _derive_coef(R, C, s, G, A1, A2, v_ref, coef_ref):
    gl, bl = v_ref[0:1, :], v_ref[1:2, :]
    gg, bg = v_ref[2:3, :], v_ref[3:4, :]
    ab1, ag1, abt1 = v_ref[4:5, :], v_ref[5:6, :], v_ref[6:7, :]
    ab2, ag2, abt2 = v_ref[7:8, :], v_ref[8:9, :], v_ref[9:10, :]

    mu = s / R
    outer = lax.dot_general(mu, mu, (((0,), (0,)), ((), ())),
                            preferred_element_type=F32)
    cov = G / R - outer
    i0 = lax.broadcasted_iota(jnp.int32, (2 * C, 2 * C), 0)
    i1 = lax.broadcasted_iota(jnp.int32, (2 * C, 2 * C), 1)
    diag = jnp.sum(jnp.where(i0 == i1, G, 0.0), axis=0,
                   keepdims=True) / R
    var_z = diag - mu * mu
    a1 = gl * lax.rsqrt(var_z[:, :C] + EPS)
    c1 = bl - a1 * mu[:, :C]
    a2 = gg * lax.rsqrt(var_z[:, C:] + EPS)
    c2 = bg - a2 * mu[:, C:]

    mut1 = _dotT(bl + bg, A1) + ab1
    M1 = jnp.concatenate([A1 * a1, A1 * a2], axis=1)
    M1cov = lax.dot_general(M1, cov, (((1,), (0,)), ((), ())),
                            preferred_element_type=F32)
    vart1 = jnp.sum(M1cov * M1, axis=1).reshape(1, C)
    al1 = ag1 * lax.rsqrt(vart1 + EPS)
    g1c = abt1 - al1 * mut1

    mut2 = _dotT(abt1, A2) + ab2
    M2 = lax.dot_general(A2 * al1, M1, (((1,), (0,)), ((), ())),
                         preferred_element_type=F32)
    M2cov = lax.dot_general(M2, cov, (((1,), (0,)), ((), ())),
                            preferred_element_type=F32)
    vart2 = jnp.sum(M2cov * M2, axis=1).reshape(1, C)
    al2 = ag2 * lax.rsqrt(vart2 + EPS)
    g2c = abt2 - al2 * mut2

    coef_ref[0:1, :] = a1
    coef_ref[1:2, :] = c1
    coef_ref[2:3, :] = a2
    coef_ref[3:4, :] = c2
    coef_ref[4:5, :] = ab1
    coef_ref[5:6, :] = al1
    coef_ref[6:7, :] = g1c
    coef_ref[7:8, :] = ab2
    coef_ref[8:9, :] = al2
    coef_ref[9:10, :] = g2c


def _k5_body(R, C, NB, RB, xk_ref, p_ref, y2_ref, wl_ref, a1w_ref, a2w_ref,
             v_ref, gb_ref, res_ref, y1_s, o_s, s_s, g_s, coef_s,
             so_s, soq_s):
    i = pl.program_id(0)

    @pl.when(i < NB)
    def _():
        y1 = _dotT(xk_ref[...] - p_ref[...], wl_ref[...])
        y1_s[pl.ds(i * RB, RB), :] = y1
        z = jnp.concatenate([y1, y2_ref[...]], axis=1)

        @pl.when(i == 0)
        def _():
            s_s[...] = jnp.zeros_like(s_s)
            g_s[...] = jnp.zeros_like(g_s)

        s_s[...] += jnp.sum(z, axis=0, keepdims=True)
        g_s[...] += lax.dot_general(z, z, (((0,), (0,)), ((), ())),
                                    preferred_element_type=F32)

    @pl.when(i == NB)
    def _():
        _derive_coef(float(R), C, s_s[...], g_s[...], a1w_ref[...],
                     a2w_ref[...], v_ref, coef_s)

    @pl.when(jnp.logical_and(i >= NB, i < 2 * NB))
    def _():
        j = i - NB
        y1 = y1_s[pl.ds(j * RB, RB), :]
        y2 = y2_ref[...]
        cf = coef_s[...]
        a1, c1 = cf[0:1, :], cf[1:2, :]
        a2, c2 = cf[2:3, :], cf[3:4, :]
        b1, al1, g1c = cf[4:5, :], cf[5:6, :], cf[6:7, :]
        b2, al2, g2c = cf[7:8, :], cf[8:9, :], cf[9:10, :]

        x1 = y1 * a1 + c1
        x2 = y2 * a2 + c2
        t1 = _dotT(x1 + x2, a1w_ref[...]) + b1
        u1 = t1 * al1 + g1c
        t2 = _dotT(u1, a2w_ref[...]) + b2
        att = jax.nn.sigmoid(t2 * al2 + g2c)
        o = x2 + att * (x1 - x2)
        o_s[pl.ds(j * RB, RB), :] = o

        @pl.when(i == NB)
        def _():
            so_s[...] = jnp.zeros_like(so_s)
            soq_s[...] = jnp.zeros_like(soq_s)

        so_s[...] += jnp.sum(o, axis=0, keepdims=True)
        soq_s[...] += jnp.sum(o * o, axis=0, keepdims=True)

    @pl.when(i >= 2 * NB)
    def _():
        j = i - 2 * NB
        mu = so_s[...] / R
        var = soq_s[...] / R - mu * mu
        scale = gb_ref[0:1, :] * lax.rsqrt(var + EPS)
        res_ref[...] = (o_s[pl.ds(j * RB, RB), :] - mu) * scale \
            + gb_ref[1:2, :]


def _run_k5(xk, p, y2, w_local, aff_w1, aff_w2, vecs, gb, rb):
    r, c = xk.shape
    nb = r // rb
    row1 = pl.BlockSpec((rb, c), lambda i: (jnp.minimum(i, nb - 1), 0))
    row12 = pl.BlockSpec(
        (rb, c),
        lambda i: (jnp.where(i < nb, i, jnp.minimum(i - nb, nb - 1)), 0))
    row3 = pl.BlockSpec((rb, c), lambda i: (jnp.maximum(i - 2 * nb, 0), 0))
    w_spec = pl.BlockSpec((c, c), lambda i: (0, 0))
    return pl.pallas_call(
        functools.partial(_k5_body, r, c, nb, rb),
        grid=(3 * nb,),
        in_specs=[row1, row1, row12, w_spec, w_spec, w_spec,
                  pl.BlockSpec((10, c), lambda i: (0, 0)),
                  pl.BlockSpec((2, c), lambda i: (0, 0))],
        out_specs=row3,
        out_shape=jax.ShapeDtypeStruct((r, c), F32),
        scratch_shapes=[
            pltpu.VMEM((r, c), F32),
            pltpu.VMEM((r, c), F32),
            pltpu.VMEM((1, 2 * c), F32),
            pltpu.VMEM((2 * c, 2 * c), F32),
            pltpu.VMEM((16, c), F32),
            pltpu.VMEM((1, c), F32),
            pltpu.VMEM((1, c), F32),
        ],
    )(xk, p, y2, w_local, aff_w1, aff_w2, vecs, gb)


def kernel(x, knn, w_proj, w_local, g_local, b_local, w_global, g_global,
           b_global, aff_w1, aff_b1, aff_g1, aff_bt1, aff_w2, aff_b2,
           aff_g2, aff_bt2, bn_g, bn_b):
    B, N, C = x.shape
    K = knn.shape[-1]
    R = B * N
    RB = 2000

    xr = x.reshape(R, C)
    knn2 = knn.reshape(R * K // 128, 128)
    p, y2, adj = _run_k1(xr, w_proj, w_global, knn2, N, RB)

    sc = _make_sc_gather_max(R, C, K)
    xkmax = sc(p, adj.reshape(R * K))

    vecs = jnp.stack([g_local, b_local, g_global, b_global,
                      aff_b1, aff_g1, aff_bt1,
                      aff_b2, aff_g2, aff_bt2], axis=0)
    gb = jnp.stack([bn_g, bn_b], axis=0)
    res = _run_k5(xkmax, p, y2, w_local, aff_w1, aff_w2, vecs, gb, RB)
    return res.reshape(B, N, C)
